# Initial kernel scaffold; baseline (speedup 1.0000x reference)
import jax
import jax.numpy as jnp
from jax.experimental import pallas as pl


def kernel(x, y, edge_index, edge_attr, batch, emb, We, be, W1, b1, W2, b2, Wf1, bf1, Wf2, bf2, Wb1, bb1, Wb2, bb2):
    raise NotImplementedError("write your pallas kernel here")


# speedup vs baseline: 1.0153x; 1.0153x over previous
import functools

import jax
import jax.numpy as jnp
from jax import lax
from jax.experimental import pallas as pl
from jax.experimental.pallas import tpu as pltpu

N = 10000
E = 320000
NT = 10000
EMB = 128
DH = 256
NG = 512
DIN = EMB + 1

ROW_BLK = 256
N_BLKS = (N + ROW_BLK - 1) // ROW_BLK
COL_BLK = 1024
NT_BLKS = (NT + COL_BLK - 1) // COL_BLK


def _mlp_pool_body(out_ref, w1_ref, b1_ref, w2_ref, b2_ref, batch_ref, hg_ref):
    i = pl.program_id(0)
    o = out_ref[...]
    h = jnp.maximum(jnp.dot(o, w1_ref[...], preferred_element_type=jnp.float32)
                    + b1_ref[...], 0.0)
    h = jnp.maximum(jnp.dot(h, w2_ref[...], preferred_element_type=jnp.float32)
                    + b2_ref[...], 0.0)
    row = i * ROW_BLK + lax.broadcasted_iota(jnp.int32, (ROW_BLK, 1), 0)
    h = jnp.where(row < N, h, 0.0)
    b = batch_ref[0, 0, :]
    onehot_t = (lax.broadcasted_iota(jnp.int32, (NG, ROW_BLK), 0)
                == b[None, :]).astype(jnp.float32)

    @pl.when(i == 0)
    def _():
        hg_ref[...] = jnp.zeros_like(hg_ref)

    hg_ref[...] += jnp.dot(onehot_t, h, preferred_element_type=jnp.float32)


def _heads_body(hg_ref, wf1_ref, bf1_ref, wb1_ref, bb1_ref,
                wf2_ref, bf2_ref, wb2_ref, bb2_ref,
                f_ref, bck_ref, hf_s, hb_s):
    i = pl.program_id(0)

    @pl.when(i == 0)
    def _():
        hg = hg_ref[...]
        hf_s[...] = jnp.maximum(
            jnp.dot(hg, wf1_ref[...], preferred_element_type=jnp.float32)
            + bf1_ref[...], 0.0)
        hb_s[...] = jnp.maximum(
            jnp.dot(hg, wb1_ref[...], preferred_element_type=jnp.float32)
            + bb1_ref[...], 0.0)

    f_ref[...] = jnp.dot(hf_s[...], wf2_ref[...],
                         preferred_element_type=jnp.float32) + bf2_ref[...]
    bck_ref[...] = jnp.dot(hb_s[...], wb2_ref[...],
                           preferred_element_type=jnp.float32) + bb2_ref[...]


def kernel(x, y, edge_index, edge_attr, batch, emb, We, be, W1, b1, W2, b2,
           Wf1, bf1, Wf2, bf2, Wb1, bb1, Wb2, bb2):
    xe = jnp.take(emb, x[:, 0], axis=0)
    xy = jnp.concatenate([xe, y[:, None]], axis=1)
    src = edge_index[0]
    dst = edge_index[1]
    e = edge_attr @ We + be
    m = jax.nn.relu(xy[src] + e)
    agg = jax.ops.segment_sum(m, dst, num_segments=N)
    out = xy + agg

    batch_pad = jnp.full((N_BLKS * ROW_BLK,), NG, jnp.int32)
    batch_pad = batch_pad.at[:N].set(batch.astype(jnp.int32))
    batch3 = batch_pad.reshape(N_BLKS, 1, ROW_BLK)

    hg = pl.pallas_call(
        _mlp_pool_body,
        grid=(N_BLKS,),
        in_specs=[
            pl.BlockSpec((ROW_BLK, DIN), lambda i: (i, 0)),
            pl.BlockSpec((DIN, DH), lambda i: (0, 0)),
            pl.BlockSpec((1, DH), lambda i: (0, 0)),
            pl.BlockSpec((DH, DH), lambda i: (0, 0)),
            pl.BlockSpec((1, DH), lambda i: (0, 0)),
            pl.BlockSpec((1, 1, ROW_BLK), lambda i: (i, 0, 0)),
        ],
        out_specs=pl.BlockSpec((NG, DH), lambda i: (0, 0)),
        out_shape=jax.ShapeDtypeStruct((NG, DH), jnp.float32),
    )(out, W1, b1.reshape(1, DH), W2, b2.reshape(1, DH), batch3)

    f, bck = pl.pallas_call(
        _heads_body,
        grid=(NT_BLKS,),
        in_specs=[
            pl.BlockSpec((NG, DH), lambda i: (0, 0)),
            pl.BlockSpec((DH, DH), lambda i: (0, 0)),
            pl.BlockSpec((1, DH), lambda i: (0, 0)),
            pl.BlockSpec((DH, DH), lambda i: (0, 0)),
            pl.BlockSpec((1, DH), lambda i: (0, 0)),
            pl.BlockSpec((DH, COL_BLK), lambda i: (0, i)),
            pl.BlockSpec((1, COL_BLK), lambda i: (0, i)),
            pl.BlockSpec((DH, COL_BLK), lambda i: (0, i)),
            pl.BlockSpec((1, COL_BLK), lambda i: (0, i)),
        ],
        out_specs=[
            pl.BlockSpec((NG, COL_BLK), lambda i: (0, i)),
            pl.BlockSpec((NG, COL_BLK), lambda i: (0, i)),
        ],
        out_shape=[
            jax.ShapeDtypeStruct((NG, NT), jnp.float32),
            jax.ShapeDtypeStruct((NG, NT), jnp.float32),
        ],
        scratch_shapes=[
            pltpu.VMEM((NG, DH), jnp.float32),
            pltpu.VMEM((NG, DH), jnp.float32),
        ],
    )(hg, Wf1, bf1.reshape(1, DH), Wb1, bb1.reshape(1, DH),
      Wf2, bf2.reshape(1, NT), Wb2, bb2.reshape(1, NT))

    return (f, bck)


# speedup vs baseline: 3.0812x; 3.0348x over previous
import functools

import jax
import jax.numpy as jnp
from jax import lax
from jax.experimental import pallas as pl
from jax.experimental.pallas import tpu as pltpu
from jax.experimental.pallas import tpu_sc as plsc

N = 10000
E = 320000
NT = 10000
EMB = 128
DH = 256
NG = 512
DIN = EMB + 1

NPAD = 10240
DPAD = 144
NCH = 9

NW = 32
BPW = NPAD // NW
EPW = E // NW
ECHUNK = 400
NCHUNK = EPW // ECHUNK
SUB = 80

ROW_BLK = 256
N_BLKS = NPAD // ROW_BLK
COL_BLK = 1024
NT_BLKS = (NT + COL_BLK - 1) // COL_BLK

_mesh = plsc.VectorSubcoreMesh(core_axis_name="c", subcore_axis_name="s")
_sc_params = pltpu.CompilerParams(use_tc_tiling_on_sc=False)


def _xy_body(x_hbm, y_hbm, emb_hbm, xy_hbm, idxv, yv, gbuf, rows):
    cid = lax.axis_index("c")
    sid = lax.axis_index("s")
    wid = sid * 2 + cid
    pltpu.sync_copy(x_hbm.at[pl.ds(wid * BPW, BPW)], idxv)
    pltpu.sync_copy(y_hbm.at[pl.ds(wid * BPW, BPW)], yv)
    for j in range(BPW // SUB):
        pltpu.sync_copy(emb_hbm.at[idxv.at[pl.ds(j * SUB, SUB)]],
                        gbuf.at[pl.ds(j * SUB, SUB)])
    lane = lax.iota(jnp.int32, 16)

    def body(g, _):
        y16 = yv[pl.ds(g * 16, 16)]
        for l in range(16):
            r = g * 16 + l
            for c in range(EMB // 16):
                rows[r, pl.ds(c * 16, 16)] = gbuf[r, pl.ds(c * 16, 16)]
            rows[r, pl.ds(EMB, 16)] = jnp.where(
                lane == 0, jnp.full((16,), y16[l]), 0.0)
        return _

    lax.fori_loop(0, BPW // 16, body, None)
    pltpu.sync_copy(rows, xy_hbm.at[pl.ds(wid * BPW, BPW)])


_xy_kernel = functools.partial(
    pl.kernel,
    out_type=jax.ShapeDtypeStruct((NPAD, DPAD), jnp.float32),
    mesh=_mesh,
    scratch_types=[
        pltpu.VMEM((BPW,), jnp.int32),
        pltpu.VMEM((BPW,), jnp.float32),
        pltpu.VMEM((BPW, EMB), jnp.float32),
        pltpu.VMEM((BPW, DPAD), jnp.float32),
    ],
    compiler_params=_sc_params,
)(_xy_body)


NHALF = NPAD // 2
AGGR = NHALF + 128
TROWS = AGGR // 16
EPT = E // 16
NCHUNK_T = EPT // ECHUNK


def _edge_body(xy_hbm, src_hbm, dst_hbm, ea_hbm, w_hbm, b_hbm, agg_hbm,
               srcv, dstv, eav, dst80, rows, wv, bv, aggs):
    cid = lax.axis_index("c")
    sid = lax.axis_index("s")
    pltpu.sync_copy(w_hbm, wv)
    pltpu.sync_copy(b_hbm, bv)

    def zbody(r, _):
        for c in range(NCH):
            rows[r, pl.ds(c * 16, 16)] = jnp.zeros((16,), jnp.float32)
        return _

    lax.fori_loop(0, TROWS, zbody, None)
    pltpu.sync_copy(rows.at[pl.ds(0, TROWS)], aggs.at[pl.ds(sid * TROWS, TROWS)])
    plsc.subcore_barrier()

    wvec = [wv[pl.ds(c * 16, 16)] for c in range(NCH)]
    bvec = [bv[pl.ds(c * 16, 16)] for c in range(NCH)]
    lo = cid * NHALF

    def chunk(i, _):
        base = sid * EPT + i * ECHUNK
        pltpu.sync_copy(src_hbm.at[pl.ds(base, ECHUNK)], srcv)
        pltpu.sync_copy(dst_hbm.at[pl.ds(base, ECHUNK)], dstv)
        pltpu.sync_copy(ea_hbm.at[pl.ds(base, ECHUNK)], eav)
        for j in range(ECHUNK // SUB):
            pltpu.sync_copy(xy_hbm.at[srcv.at[pl.ds(j * SUB, SUB)]],
                            rows.at[pl.ds(j * SUB, SUB)])

        def ebody(g, _):
            ea16 = eav[pl.ds(g * 16, 16)]
            for l in range(16):
                rk = g * 16 + l
                eab = jnp.full((16,), ea16[l])
                for c in range(NCH):
                    r = rows[rk, pl.ds(c * 16, 16)]
                    rows[rk, pl.ds(c * 16, 16)] = jnp.maximum(
                        r + (eab * wvec[c] + bvec[c]), 0.0)
            return _

        lax.fori_loop(0, ECHUNK // 16, ebody, None)
        for j in range(ECHUNK // SUB):
            for k in range(SUB // 16):
                d16 = dstv[pl.ds(j * SUB + k * 16, 16)] - lo
                ok = (d16 >= 0) & (d16 < NHALF)
                dst80[pl.ds(k * 16, 16)] = jnp.where(ok, d16, NHALF)
            pltpu.sync_copy(rows.at[pl.ds(j * SUB, SUB)],
                            aggs.at[dst80], add=True)
        return _

    lax.fori_loop(0, NCHUNK_T, chunk, None)
    plsc.subcore_barrier()
    orows = NHALF // 16
    pltpu.sync_copy(aggs.at[pl.ds(sid * orows, orows)],
                    agg_hbm.at[pl.ds(lo + sid * orows, orows)])


_edge_kernel = functools.partial(
    pl.kernel,
    out_type=jax.ShapeDtypeStruct((NPAD, DPAD), jnp.float32),
    mesh=_mesh,
    scratch_types=[
        pltpu.VMEM((ECHUNK,), jnp.int32),
        pltpu.VMEM((ECHUNK,), jnp.int32),
        pltpu.VMEM((ECHUNK,), jnp.float32),
        pltpu.VMEM((SUB,), jnp.int32),
        pltpu.VMEM((ECHUNK, DPAD), jnp.float32),
        pltpu.VMEM((DPAD,), jnp.float32),
        pltpu.VMEM((DPAD,), jnp.float32),
        pltpu.VMEM_SHARED((AGGR, DPAD), jnp.float32),
    ],
    compiler_params=_sc_params,
)(_edge_body)


def _mlp_pool_body(xy_ref, agg_ref, w1_ref, b1_ref, w2_ref, b2_ref,
                   batch_ref, hg_ref):
    i = pl.program_id(0)
    o = xy_ref[...] + agg_ref[...]
    h = jnp.maximum(jnp.dot(o, w1_ref[...], preferred_element_type=jnp.float32)
                    + b1_ref[...], 0.0)
    h = jnp.maximum(jnp.dot(h, w2_ref[...], preferred_element_type=jnp.float32)
                    + b2_ref[...], 0.0)
    b = batch_ref[0, 0, :]
    onehot_t = (lax.broadcasted_iota(jnp.int32, (NG, ROW_BLK), 0)
                == b[None, :]).astype(jnp.float32)

    @pl.when(i == 0)
    def _():
        hg_ref[...] = jnp.zeros_like(hg_ref)

    hg_ref[...] += jnp.dot(onehot_t, h, preferred_element_type=jnp.float32)


def _heads_body(hg_ref, wf1_ref, bf1_ref, wb1_ref, bb1_ref,
                wf2_ref, bf2_ref, wb2_ref, bb2_ref,
                f_ref, bck_ref, hf_s, hb_s):
    i = pl.program_id(0)

    @pl.when(i == 0)
    def _():
        hg = hg_ref[...]
        hf_s[...] = jnp.maximum(
            jnp.dot(hg, wf1_ref[...], preferred_element_type=jnp.float32)
            + bf1_ref[...], 0.0)
        hb_s[...] = jnp.maximum(
            jnp.dot(hg, wb1_ref[...], preferred_element_type=jnp.float32)
            + bb1_ref[...], 0.0)

    f_ref[...] = jnp.dot(hf_s[...], wf2_ref[...],
                         preferred_element_type=jnp.float32) + bf2_ref[...]
    bck_ref[...] = jnp.dot(hb_s[...], wb2_ref[...],
                           preferred_element_type=jnp.float32) + bb2_ref[...]


def kernel(x, y, edge_index, edge_attr, batch, emb, We, be, W1, b1, W2, b2,
           Wf1, bf1, Wf2, bf2, Wb1, bb1, Wb2, bb2):
    x_pad = jnp.zeros((NPAD,), jnp.int32).at[:N].set(x[:, 0].astype(jnp.int32))
    y_pad = jnp.zeros((NPAD,), jnp.float32).at[:N].set(y)
    xy = _xy_kernel(x_pad, y_pad, emb)

    src = edge_index[0].astype(jnp.int32)
    dst = edge_index[1].astype(jnp.int32)
    ea = edge_attr[:, 0]
    wrow = jnp.zeros((DPAD,), jnp.float32).at[:DIN].set(We[0])
    brow = jnp.zeros((DPAD,), jnp.float32).at[:DIN].set(be)
    agg2 = _edge_kernel(xy, src, dst, ea, wrow, brow)

    W1p = jnp.zeros((DPAD, DH), jnp.float32).at[:DIN].set(W1)
    batch_pad = jnp.full((NPAD,), NG, jnp.int32).at[:N].set(
        batch.astype(jnp.int32))
    batch3 = batch_pad.reshape(N_BLKS, 1, ROW_BLK)

    hg = pl.pallas_call(
        _mlp_pool_body,
        grid=(N_BLKS,),
        in_specs=[
            pl.BlockSpec((ROW_BLK, DPAD), lambda i: (i, 0)),
            pl.BlockSpec((ROW_BLK, DPAD), lambda i: (i, 0)),
            pl.BlockSpec((DPAD, DH), lambda i: (0, 0)),
            pl.BlockSpec((1, DH), lambda i: (0, 0)),
            pl.BlockSpec((DH, DH), lambda i: (0, 0)),
            pl.BlockSpec((1, DH), lambda i: (0, 0)),
            pl.BlockSpec((1, 1, ROW_BLK), lambda i: (i, 0, 0)),
        ],
        out_specs=pl.BlockSpec((NG, DH), lambda i: (0, 0)),
        out_shape=jax.ShapeDtypeStruct((NG, DH), jnp.float32),
    )(xy, agg2, W1p, b1.reshape(1, DH), W2, b2.reshape(1, DH), batch3)

    f, bck = pl.pallas_call(
        _heads_body,
        grid=(NT_BLKS,),
        in_specs=[
            pl.BlockSpec((NG, DH), lambda i: (0, 0)),
            pl.BlockSpec((DH, DH), lambda i: (0, 0)),
            pl.BlockSpec((1, DH), lambda i: (0, 0)),
            pl.BlockSpec((DH, DH), lambda i: (0, 0)),
            pl.BlockSpec((1, DH), lambda i: (0, 0)),
            pl.BlockSpec((DH, COL_BLK), lambda i: (0, i)),
            pl.BlockSpec((1, COL_BLK), lambda i: (0, i)),
            pl.BlockSpec((DH, COL_BLK), lambda i: (0, i)),
            pl.BlockSpec((1, COL_BLK), lambda i: (0, i)),
        ],
        out_specs=[
            pl.BlockSpec((NG, COL_BLK), lambda i: (0, i)),
            pl.BlockSpec((NG, COL_BLK), lambda i: (0, i)),
        ],
        out_shape=[
            jax.ShapeDtypeStruct((NG, NT), jnp.float32),
            jax.ShapeDtypeStruct((NG, NT), jnp.float32),
        ],
        scratch_shapes=[
            pltpu.VMEM((NG, DH), jnp.float32),
            pltpu.VMEM((NG, DH), jnp.float32),
        ],
    )(hg, Wf1, bf1.reshape(1, DH), Wb1, bb1.reshape(1, DH),
      Wf2, bf2.reshape(1, NT), Wb2, bb2.reshape(1, NT))

    return (f, bck)


# speedup vs baseline: 5.0419x; 1.6364x over previous
import functools

import jax
import jax.numpy as jnp
from jax import lax
from jax.experimental import pallas as pl
from jax.experimental.pallas import tpu as pltpu
from jax.experimental.pallas import tpu_sc as plsc

N = 10000
E = 320000
NT = 10000
EMB = 128
DH = 256
NG = 512
DIN = EMB + 1

NPAD = 10240
DPAD = 144
NCH = 9

NW = 32
BPW = NPAD // NW
EPW = E // NW
ECHUNK = 80
NCHUNK = EPW // ECHUNK
SUB = 80

ROW_BLK = 256
N_BLKS = NPAD // ROW_BLK
COL_BLK = 1024
NT_BLKS = (NT + COL_BLK - 1) // COL_BLK

_mesh = plsc.VectorSubcoreMesh(core_axis_name="c", subcore_axis_name="s")
_sc_params = pltpu.CompilerParams(use_tc_tiling_on_sc=False)


def _xy_body(x_hbm, y_hbm, emb_hbm, xy_hbm, idxv, yv, gbuf, rows):
    cid = lax.axis_index("c")
    sid = lax.axis_index("s")
    wid = sid * 2 + cid
    pltpu.sync_copy(x_hbm.at[pl.ds(wid * BPW, BPW)], idxv)
    pltpu.sync_copy(y_hbm.at[pl.ds(wid * BPW, BPW)], yv)
    for j in range(BPW // SUB):
        pltpu.sync_copy(emb_hbm.at[idxv.at[pl.ds(j * SUB, SUB)]],
                        gbuf.at[pl.ds(j * SUB, SUB)])
    lane = lax.iota(jnp.int32, 16)

    def body(g, _):
        y16 = yv[pl.ds(g * 16, 16)]
        for l in range(16):
            r = g * 16 + l
            for c in range(EMB // 16):
                rows[r, pl.ds(c * 16, 16)] = gbuf[r, pl.ds(c * 16, 16)]
            rows[r, pl.ds(EMB, 16)] = jnp.where(
                lane == 0, jnp.full((16,), y16[l]), 0.0)
        return _

    lax.fori_loop(0, BPW // 16, body, None)
    pltpu.sync_copy(rows, xy_hbm.at[pl.ds(wid * BPW, BPW)])


_xy_kernel = functools.partial(
    pl.kernel,
    out_type=jax.ShapeDtypeStruct((NPAD, DPAD), jnp.float32),
    mesh=_mesh,
    scratch_types=[
        pltpu.VMEM((BPW,), jnp.int32),
        pltpu.VMEM((BPW,), jnp.float32),
        pltpu.VMEM((BPW, EMB), jnp.float32),
        pltpu.VMEM((BPW, DPAD), jnp.float32),
    ],
    compiler_params=_sc_params,
)(_xy_body)


NHALF = NPAD // 2
AGGR = NHALF + 128
TROWS = AGGR // 16
EPT = E // 16
NCHUNK_T = EPT // ECHUNK


NSUB = ECHUNK // SUB


def _edge_body(xy_hbm, src_hbm, dst_hbm, ea_hbm, w_hbm, b_hbm, agg_hbm,
               srcv, dstv, eav, dsti, rows, wv, bv, aggs,
               isem0, isem1, gsem0, gsem1, ssem0, ssem1):
    cid = lax.axis_index("c")
    sid = lax.axis_index("s")
    pltpu.sync_copy(w_hbm, wv)
    pltpu.sync_copy(b_hbm, bv)
    isem = (isem0, isem1)
    gsem = (gsem0, gsem1)
    ssem = (ssem0, ssem1)

    def zbody(r, _):
        for c in range(NCH):
            rows[0, r, pl.ds(c * 16, 16)] = jnp.zeros((16,), jnp.float32)
        return _

    lax.fori_loop(0, ECHUNK, zbody, None)
    for p in range(TROWS // ECHUNK):
        pltpu.sync_copy(rows.at[0],
                        aggs.at[pl.ds(sid * TROWS + p * ECHUNK, ECHUNK)])
    rem = TROWS % ECHUNK
    if rem:
        pltpu.sync_copy(
            rows.at[0, pl.ds(0, rem)],
            aggs.at[pl.ds(sid * TROWS + TROWS - rem, rem)])
    plsc.subcore_barrier()

    wvec = [wv[pl.ds(c * 16, 16)] for c in range(NCH)]
    bvec = [bv[pl.ds(c * 16, 16)] for c in range(NCH)]
    lo = cid * NHALF

    def idx_copies(i, b):
        base = sid * EPT + i * ECHUNK
        return (
            pltpu.make_async_copy(src_hbm.at[pl.ds(base, ECHUNK)],
                                  srcv.at[b], isem[b]),
            pltpu.make_async_copy(dst_hbm.at[pl.ds(base, ECHUNK)],
                                  dstv.at[b], isem[b]),
            pltpu.make_async_copy(ea_hbm.at[pl.ds(base, ECHUNK)],
                                  eav.at[b], isem[b]),
        )

    def gather_copies(b):
        return tuple(
            pltpu.make_async_copy(
                xy_hbm.at[srcv.at[b, pl.ds(j * SUB, SUB)]],
                rows.at[b, pl.ds(j * SUB, SUB)], gsem[b])
            for j in range(NSUB))

    def scatter_start(b):
        for j in range(NSUB):
            pltpu.async_copy(rows.at[b, pl.ds(j * SUB, SUB)],
                             aggs.at[dsti.at[b, j]], ssem[b], add=True)

    def scatter_wait(b):
        for j in range(NSUB):
            pltpu.make_async_copy(rows.at[b, pl.ds(j * SUB, SUB)],
                                  aggs.at[dsti.at[b, j]], ssem[b]).wait()

    def start(copies):
        for c in copies:
            c.start()

    def wait(copies):
        for c in copies:
            c.wait()

    def compute(i, b):
        def ebody(g, _):
            ea16 = eav[b, pl.ds(g * 16, 16)]
            for l in range(16):
                rk = g * 16 + l
                eab = jnp.full((16,), ea16[l])
                for c in range(NCH):
                    r = rows[b, rk, pl.ds(c * 16, 16)]
                    rows[b, rk, pl.ds(c * 16, 16)] = jnp.maximum(
                        r + (eab * wvec[c] + bvec[c]), 0.0)
            return _

        lax.fori_loop(0, ECHUNK // 16, ebody, None)
        for j in range(NSUB):
            for k in range(SUB // 16):
                d16 = dstv[b, pl.ds(j * SUB + k * 16, 16)] - lo
                ok = (d16 >= 0) & (d16 < NHALF)
                dsti[b, j, pl.ds(k * 16, 16)] = jnp.where(ok, d16, NHALF)

    start(idx_copies(0, 0))
    start(idx_copies(1, 1))
    wait(idx_copies(0, 0))
    start(gather_copies(0))

    def step(i, b, nb):
        @pl.when(i + 1 < NCHUNK_T)
        def _():
            wait(idx_copies(i + 1, nb))

        @pl.when(i >= 1)
        def _():
            scatter_wait(nb)

        @pl.when(i + 1 < NCHUNK_T)
        def _():
            start(gather_copies(nb))

        wait(gather_copies(b))
        compute(i, b)

        @pl.when(i + 2 < NCHUNK_T)
        def _():
            start(idx_copies(i + 2, b))

        scatter_start(b)

    def pair(it, _):
        step(2 * it, 0, 1)
        step(2 * it + 1, 1, 0)
        return _

    lax.fori_loop(0, NCHUNK_T // 2, pair, None)
    scatter_wait((NCHUNK_T - 1) % 2)
    plsc.subcore_barrier()
    orows = NHALF // 16
    pltpu.sync_copy(aggs.at[pl.ds(sid * orows, orows)],
                    agg_hbm.at[pl.ds(lo + sid * orows, orows)])


_edge_kernel = functools.partial(
    pl.kernel,
    out_type=jax.ShapeDtypeStruct((NPAD, DPAD), jnp.float32),
    mesh=_mesh,
    scratch_types=[
        pltpu.VMEM((2, ECHUNK), jnp.int32),
        pltpu.VMEM((2, ECHUNK), jnp.int32),
        pltpu.VMEM((2, ECHUNK), jnp.float32),
        pltpu.VMEM((2, NSUB, SUB), jnp.int32),
        pltpu.VMEM((2, ECHUNK, DPAD), jnp.float32),
        pltpu.VMEM((DPAD,), jnp.float32),
        pltpu.VMEM((DPAD,), jnp.float32),
        pltpu.VMEM_SHARED((AGGR, DPAD), jnp.float32),
        pltpu.SemaphoreType.DMA,
        pltpu.SemaphoreType.DMA,
        pltpu.SemaphoreType.DMA,
        pltpu.SemaphoreType.DMA,
        pltpu.SemaphoreType.DMA,
        pltpu.SemaphoreType.DMA,
    ],
    compiler_params=_sc_params,
)(_edge_body)


def _mlp_pool_body(xy_ref, agg_ref, w1_ref, b1_ref, w2_ref, b2_ref,
                   batch_ref, hg_ref):
    i = pl.program_id(0)
    o = xy_ref[...] + agg_ref[...]
    h = jnp.maximum(jnp.dot(o, w1_ref[...], preferred_element_type=jnp.float32)
                    + b1_ref[...], 0.0)
    h = jnp.maximum(jnp.dot(h, w2_ref[...], preferred_element_type=jnp.float32)
                    + b2_ref[...], 0.0)
    b = batch_ref[0, 0, :]
    onehot_t = (lax.broadcasted_iota(jnp.int32, (NG, ROW_BLK), 0)
                == b[None, :]).astype(jnp.float32)

    @pl.when(i == 0)
    def _():
        hg_ref[...] = jnp.zeros_like(hg_ref)

    hg_ref[...] += jnp.dot(onehot_t, h, preferred_element_type=jnp.float32)


def _heads_body(hg_ref, wf1_ref, bf1_ref, wb1_ref, bb1_ref,
                wf2_ref, bf2_ref, wb2_ref, bb2_ref,
                f_ref, bck_ref, hf_s, hb_s):
    i = pl.program_id(0)

    @pl.when(i == 0)
    def _():
        hg = hg_ref[...]
        hf_s[...] = jnp.maximum(
            jnp.dot(hg, wf1_ref[...], preferred_element_type=jnp.float32)
            + bf1_ref[...], 0.0)
        hb_s[...] = jnp.maximum(
            jnp.dot(hg, wb1_ref[...], preferred_element_type=jnp.float32)
            + bb1_ref[...], 0.0)

    f_ref[...] = jnp.dot(hf_s[...], wf2_ref[...],
                         preferred_element_type=jnp.float32) + bf2_ref[...]
    bck_ref[...] = jnp.dot(hb_s[...], wb2_ref[...],
                           preferred_element_type=jnp.float32) + bb2_ref[...]


def kernel(x, y, edge_index, edge_attr, batch, emb, We, be, W1, b1, W2, b2,
           Wf1, bf1, Wf2, bf2, Wb1, bb1, Wb2, bb2):
    x_pad = jnp.zeros((NPAD,), jnp.int32).at[:N].set(x[:, 0].astype(jnp.int32))
    y_pad = jnp.zeros((NPAD,), jnp.float32).at[:N].set(y)
    xy = _xy_kernel(x_pad, y_pad, emb)

    src = edge_index[0].astype(jnp.int32)
    dst = edge_index[1].astype(jnp.int32)
    ea = edge_attr[:, 0]
    wrow = jnp.zeros((DPAD,), jnp.float32).at[:DIN].set(We[0])
    brow = jnp.zeros((DPAD,), jnp.float32).at[:DIN].set(be)
    agg2 = _edge_kernel(xy, src, dst, ea, wrow, brow)

    W1p = jnp.zeros((DPAD, DH), jnp.float32).at[:DIN].set(W1)
    batch_pad = jnp.full((NPAD,), NG, jnp.int32).at[:N].set(
        batch.astype(jnp.int32))
    batch3 = batch_pad.reshape(N_BLKS, 1, ROW_BLK)

    hg = pl.pallas_call(
        _mlp_pool_body,
        grid=(N_BLKS,),
        in_specs=[
            pl.BlockSpec((ROW_BLK, DPAD), lambda i: (i, 0)),
            pl.BlockSpec((ROW_BLK, DPAD), lambda i: (i, 0)),
            pl.BlockSpec((DPAD, DH), lambda i: (0, 0)),
            pl.BlockSpec((1, DH), lambda i: (0, 0)),
            pl.BlockSpec((DH, DH), lambda i: (0, 0)),
            pl.BlockSpec((1, DH), lambda i: (0, 0)),
            pl.BlockSpec((1, 1, ROW_BLK), lambda i: (i, 0, 0)),
        ],
        out_specs=pl.BlockSpec((NG, DH), lambda i: (0, 0)),
        out_shape=jax.ShapeDtypeStruct((NG, DH), jnp.float32),
    )(xy, agg2, W1p, b1.reshape(1, DH), W2, b2.reshape(1, DH), batch3)

    f, bck = pl.pallas_call(
        _heads_body,
        grid=(NT_BLKS,),
        in_specs=[
            pl.BlockSpec((NG, DH), lambda i: (0, 0)),
            pl.BlockSpec((DH, DH), lambda i: (0, 0)),
            pl.BlockSpec((1, DH), lambda i: (0, 0)),
            pl.BlockSpec((DH, DH), lambda i: (0, 0)),
            pl.BlockSpec((1, DH), lambda i: (0, 0)),
            pl.BlockSpec((DH, COL_BLK), lambda i: (0, i)),
            pl.BlockSpec((1, COL_BLK), lambda i: (0, i)),
            pl.BlockSpec((DH, COL_BLK), lambda i: (0, i)),
            pl.BlockSpec((1, COL_BLK), lambda i: (0, i)),
        ],
        out_specs=[
            pl.BlockSpec((NG, COL_BLK), lambda i: (0, i)),
            pl.BlockSpec((NG, COL_BLK), lambda i: (0, i)),
        ],
        out_shape=[
            jax.ShapeDtypeStruct((NG, NT), jnp.float32),
            jax.ShapeDtypeStruct((NG, NT), jnp.float32),
        ],
        scratch_shapes=[
            pltpu.VMEM((NG, DH), jnp.float32),
            pltpu.VMEM((NG, DH), jnp.float32),
        ],
    )(hg, Wf1, bf1.reshape(1, DH), Wb1, bb1.reshape(1, DH),
      Wf2, bf2.reshape(1, NT), Wb2, bb2.reshape(1, NT))

    return (f, bck)


# speedup vs baseline: 6.0901x; 1.2079x over previous
import functools

import jax
import jax.numpy as jnp
from jax import lax
from jax.experimental import pallas as pl
from jax.experimental.pallas import tpu as pltpu
from jax.experimental.pallas import tpu_sc as plsc

N = 10000
E = 320000
NT = 10000
EMB = 128
DH = 256
NG = 512
DIN = EMB + 1

NPAD = 10240
DPAD = 160
HALF = DPAD // 2
NCHH = HALF // 16

NW = 32
BPW = NPAD // NW
ECHUNK = 80
EPT = E // 16
NCHUNK_T = EPT // ECHUNK

ROW_BLK = 256
N_BLKS = NPAD // ROW_BLK
COL_BLK = 1024
NT_BLKS = (NT + COL_BLK - 1) // COL_BLK

_mesh = plsc.VectorSubcoreMesh(core_axis_name="c", subcore_axis_name="s")
_sc_params = pltpu.CompilerParams(use_tc_tiling_on_sc=False)


def _xy_body(x_hbm, y_hbm, emb_hbm, xy_hbm, idxv, yv, gbuf, rowsa, rowsb):
    cid = lax.axis_index("c")
    sid = lax.axis_index("s")
    wid = sid * 2 + cid
    pltpu.sync_copy(x_hbm.at[pl.ds(wid * BPW, BPW)], idxv)
    pltpu.sync_copy(y_hbm.at[pl.ds(wid * BPW, BPW)], yv)
    for j in range(BPW // 80):
        pltpu.sync_copy(emb_hbm.at[idxv.at[pl.ds(j * 80, 80)]],
                        gbuf.at[pl.ds(j * 80, 80)])
    lane = lax.iota(jnp.int32, 16)
    zero16 = jnp.zeros((16,), jnp.float32)

    def body(g, _):
        y16 = yv[pl.ds(g * 16, 16)]
        for l in range(16):
            r = g * 16 + l
            for c in range(NCHH):
                rowsa[r, pl.ds(c * 16, 16)] = gbuf[r, pl.ds(c * 16, 16)]
            for c in range(3):
                rowsb[r, pl.ds(c * 16, 16)] = gbuf[r, pl.ds(80 + c * 16, 16)]
            rowsb[r, pl.ds(48, 16)] = jnp.where(
                lane == 0, jnp.full((16,), y16[l]), 0.0)
            rowsb[r, pl.ds(64, 16)] = zero16
        return _

    lax.fori_loop(0, BPW // 16, body, None)
    pltpu.sync_copy(rowsa, xy_hbm.at[pl.ds(wid * BPW, BPW)])
    pltpu.sync_copy(rowsb, xy_hbm.at[pl.ds(NPAD + wid * BPW, BPW)])


_xy_kernel = functools.partial(
    pl.kernel,
    out_type=jax.ShapeDtypeStruct((2 * NPAD, HALF), jnp.float32),
    mesh=_mesh,
    scratch_types=[
        pltpu.VMEM((BPW,), jnp.int32),
        pltpu.VMEM((BPW,), jnp.float32),
        pltpu.VMEM((BPW, EMB), jnp.float32),
        pltpu.VMEM((BPW, HALF), jnp.float32),
        pltpu.VMEM((BPW, HALF), jnp.float32),
    ],
    compiler_params=_sc_params,
)(_xy_body)


def _edge_body(xy_hbm, src_hbm, dst_hbm, ea_hbm, w_hbm, b_hbm, agg_hbm,
               srcv, dstv, eav, rows, wv, bv, aggs,
               isem0, isem1, gsem0, gsem1, ssem0, ssem1):
    cid = lax.axis_index("c")
    sid = lax.axis_index("s")
    pltpu.sync_copy(w_hbm.at[cid], wv)
    pltpu.sync_copy(b_hbm.at[cid], bv)
    isem = (isem0, isem1)
    gsem = (gsem0, gsem1)
    ssem = (ssem0, ssem1)

    def zbody(r, _):
        for c in range(NCHH):
            rows[0, r, pl.ds(c * 16, 16)] = jnp.zeros((16,), jnp.float32)
        return _

    lax.fori_loop(0, ECHUNK, zbody, None)
    trows = NPAD // 16
    for p in range(trows // ECHUNK):
        pltpu.sync_copy(rows.at[0],
                        aggs.at[pl.ds(sid * trows + p * ECHUNK, ECHUNK)])
    plsc.subcore_barrier()

    wvec = [wv[pl.ds(c * 16, 16)] for c in range(NCHH)]
    bvec = [bv[pl.ds(c * 16, 16)] for c in range(NCHH)]

    def idx_copies(i, b):
        base = sid * EPT + i * ECHUNK
        return (
            pltpu.make_async_copy(src_hbm.at[cid, pl.ds(base, ECHUNK)],
                                  srcv.at[b], isem[b]),
            pltpu.make_async_copy(dst_hbm.at[pl.ds(base, ECHUNK)],
                                  dstv.at[b], isem[b]),
            pltpu.make_async_copy(ea_hbm.at[pl.ds(base, ECHUNK)],
                                  eav.at[b], isem[b]),
        )

    def gather_copy(b):
        return pltpu.make_async_copy(xy_hbm.at[srcv.at[b]],
                                     rows.at[b], gsem[b])

    def scatter_start(b):
        pltpu.async_copy(rows.at[b], aggs.at[dstv.at[b]], ssem[b], add=True)

    def scatter_wait(b):
        pltpu.make_async_copy(rows.at[b], aggs.at[dstv.at[b]],
                              ssem[b]).wait()

    def start(copies):
        for c in copies:
            c.start()

    def wait(copies):
        for c in copies:
            c.wait()

    def compute(b):
        def ebody(g, _):
            ea16 = eav[b, pl.ds(g * 16, 16)]
            for l in range(16):
                rk = g * 16 + l
                eab = jnp.full((16,), ea16[l])
                for c in range(NCHH):
                    r = rows[b, rk, pl.ds(c * 16, 16)]
                    rows[b, rk, pl.ds(c * 16, 16)] = jnp.maximum(
                        r + (eab * wvec[c] + bvec[c]), 0.0)
            return _

        lax.fori_loop(0, ECHUNK // 16, ebody, None)

    start(idx_copies(0, 0))
    start(idx_copies(1, 1))
    wait(idx_copies(0, 0))
    gather_copy(0).start()

    def step(i, b, nb):
        @pl.when(i + 1 < NCHUNK_T)
        def _():
            wait(idx_copies(i + 1, nb))

        @pl.when(i >= 1)
        def _():
            scatter_wait(nb)

        @pl.when(i + 1 < NCHUNK_T)
        def _():
            gather_copy(nb).start()

        gather_copy(b).wait()
        compute(b)

        @pl.when(i + 2 < NCHUNK_T)
        def _():
            start(idx_copies(i + 2, b))

        scatter_start(b)

    def pair(it, _):
        step(2 * it, 0, 1)
        step(2 * it + 1, 1, 0)
        return _

    lax.fori_loop(0, NCHUNK_T // 2, pair, None)
    scatter_wait((NCHUNK_T - 1) % 2)
    plsc.subcore_barrier()
    trows2 = NPAD // 16
    pltpu.sync_copy(aggs.at[pl.ds(sid * trows2, trows2)],
                    agg_hbm.at[cid, pl.ds(sid * trows2, trows2)])


_edge_kernel = functools.partial(
    pl.kernel,
    out_type=jax.ShapeDtypeStruct((2, NPAD, HALF), jnp.float32),
    mesh=_mesh,
    scratch_types=[
        pltpu.VMEM((2, ECHUNK), jnp.int32),
        pltpu.VMEM((2, ECHUNK), jnp.int32),
        pltpu.VMEM((2, ECHUNK), jnp.float32),
        pltpu.VMEM((2, ECHUNK, HALF), jnp.float32),
        pltpu.VMEM((HALF,), jnp.float32),
        pltpu.VMEM((HALF,), jnp.float32),
        pltpu.VMEM_SHARED((NPAD, HALF), jnp.float32),
        pltpu.SemaphoreType.DMA,
        pltpu.SemaphoreType.DMA,
        pltpu.SemaphoreType.DMA,
        pltpu.SemaphoreType.DMA,
        pltpu.SemaphoreType.DMA,
        pltpu.SemaphoreType.DMA,
    ],
    compiler_params=_sc_params,
)(_edge_body)


def _mlp_pool_body(xya_ref, xyb_ref, aga_ref, agb_ref, w1a_ref, w1b_ref,
                   b1_ref, w2_ref, b2_ref, batch_ref, hg_ref):
    i = pl.program_id(0)
    oa = xya_ref[0] + aga_ref[0]
    ob = xyb_ref[0] + agb_ref[0]
    h = jnp.maximum(
        jnp.dot(oa, w1a_ref[...], preferred_element_type=jnp.float32)
        + jnp.dot(ob, w1b_ref[...], preferred_element_type=jnp.float32)
        + b1_ref[...], 0.0)
    h = jnp.maximum(jnp.dot(h, w2_ref[...], preferred_element_type=jnp.float32)
                    + b2_ref[...], 0.0)
    b = batch_ref[0, 0, :]
    onehot_t = (lax.broadcasted_iota(jnp.int32, (NG, ROW_BLK), 0)
                == b[None, :]).astype(jnp.float32)

    @pl.when(i == 0)
    def _():
        hg_ref[...] = jnp.zeros_like(hg_ref)

    hg_ref[...] += jnp.dot(onehot_t, h, preferred_element_type=jnp.float32)


def _heads_body(hg_ref, wf1_ref, bf1_ref, wb1_ref, bb1_ref,
                wf2_ref, bf2_ref, wb2_ref, bb2_ref,
                f_ref, bck_ref, hf_s, hb_s):
    i = pl.program_id(0)

    @pl.when(i == 0)
    def _():
        hg = hg_ref[...]
        hf_s[...] = jnp.maximum(
            jnp.dot(hg, wf1_ref[...], preferred_element_type=jnp.float32)
            + bf1_ref[...], 0.0)
        hb_s[...] = jnp.maximum(
            jnp.dot(hg, wb1_ref[...], preferred_element_type=jnp.float32)
            + bb1_ref[...], 0.0)

    f_ref[...] = jnp.dot(hf_s[...], wf2_ref[...],
                         preferred_element_type=jnp.float32) + bf2_ref[...]
    bck_ref[...] = jnp.dot(hb_s[...], wb2_ref[...],
                           preferred_element_type=jnp.float32) + bb2_ref[...]


def kernel(x, y, edge_index, edge_attr, batch, emb, We, be, W1, b1, W2, b2,
           Wf1, bf1, Wf2, bf2, Wb1, bb1, Wb2, bb2):
    x_pad = jnp.zeros((NPAD,), jnp.int32).at[:N].set(x[:, 0].astype(jnp.int32))
    y_pad = jnp.zeros((NPAD,), jnp.float32).at[:N].set(y)
    xycat = _xy_kernel(x_pad, y_pad, emb)

    src = edge_index[0].astype(jnp.int32)
    src2 = jnp.stack([src, src + NPAD])
    dst = edge_index[1].astype(jnp.int32)
    ea = edge_attr[:, 0]
    wrow = jnp.zeros((DPAD,), jnp.float32).at[:DIN].set(We[0]).reshape(2, HALF)
    brow = jnp.zeros((DPAD,), jnp.float32).at[:DIN].set(be).reshape(2, HALF)
    agg2 = _edge_kernel(xycat, src2, dst, ea, wrow, brow)

    W1p = jnp.zeros((DPAD, DH), jnp.float32).at[:DIN].set(W1)
    xy3 = xycat.reshape(2, NPAD, HALF)
    batch_pad = jnp.full((NPAD,), NG, jnp.int32).at[:N].set(
        batch.astype(jnp.int32))
    batch3 = batch_pad.reshape(N_BLKS, 1, ROW_BLK)

    hg = pl.pallas_call(
        _mlp_pool_body,
        grid=(N_BLKS,),
        in_specs=[
            pl.BlockSpec((1, ROW_BLK, HALF), lambda i: (0, i, 0)),
            pl.BlockSpec((1, ROW_BLK, HALF), lambda i: (1, i, 0)),
            pl.BlockSpec((1, ROW_BLK, HALF), lambda i: (0, i, 0)),
            pl.BlockSpec((1, ROW_BLK, HALF), lambda i: (1, i, 0)),
            pl.BlockSpec((HALF, DH), lambda i: (0, 0)),
            pl.BlockSpec((HALF, DH), lambda i: (0, 0)),
            pl.BlockSpec((1, DH), lambda i: (0, 0)),
            pl.BlockSpec((DH, DH), lambda i: (0, 0)),
            pl.BlockSpec((1, DH), lambda i: (0, 0)),
            pl.BlockSpec((1, 1, ROW_BLK), lambda i: (i, 0, 0)),
        ],
        out_specs=pl.BlockSpec((NG, DH), lambda i: (0, 0)),
        out_shape=jax.ShapeDtypeStruct((NG, DH), jnp.float32),
    )(xy3, xy3, agg2, agg2, W1p[:HALF], W1p[HALF:],
      b1.reshape(1, DH), W2, b2.reshape(1, DH), batch3)

    f, bck = pl.pallas_call(
        _heads_body,
        grid=(NT_BLKS,),
        in_specs=[
            pl.BlockSpec((NG, DH), lambda i: (0, 0)),
            pl.BlockSpec((DH, DH), lambda i: (0, 0)),
            pl.BlockSpec((1, DH), lambda i: (0, 0)),
            pl.BlockSpec((DH, DH), lambda i: (0, 0)),
            pl.BlockSpec((1, DH), lambda i: (0, 0)),
            pl.BlockSpec((DH, COL_BLK), lambda i: (0, i)),
            pl.BlockSpec((1, COL_BLK), lambda i: (0, i)),
            pl.BlockSpec((DH, COL_BLK), lambda i: (0, i)),
            pl.BlockSpec((1, COL_BLK), lambda i: (0, i)),
        ],
        out_specs=[
            pl.BlockSpec((NG, COL_BLK), lambda i: (0, i)),
            pl.BlockSpec((NG, COL_BLK), lambda i: (0, i)),
        ],
        out_shape=[
            jax.ShapeDtypeStruct((NG, NT), jnp.float32),
            jax.ShapeDtypeStruct((NG, NT), jnp.float32),
        ],
        scratch_shapes=[
            pltpu.VMEM((NG, DH), jnp.float32),
            pltpu.VMEM((NG, DH), jnp.float32),
        ],
    )(hg, Wf1, bf1.reshape(1, DH), Wb1, bb1.reshape(1, DH),
      Wf2, bf2.reshape(1, NT), Wb2, bb2.reshape(1, NT))

    return (f, bck)


# speedup vs baseline: 7.2891x; 1.1969x over previous
import functools

import jax
import jax.numpy as jnp
from jax import lax
from jax.experimental import pallas as pl
from jax.experimental.pallas import tpu as pltpu
from jax.experimental.pallas import tpu_sc as plsc

N = 10000
E = 320000
NT = 10000
EMB = 128
DH = 256
NG = 512
DIN = EMB + 1

NPAD = 10240
DPAD = 160
HALF = DPAD // 2
NCHH = HALF // 16

NW = 32
BPW = NPAD // NW
ECHUNK = 400
SUB = 80
NSUB = ECHUNK // SUB
EPT = E // 16
NCHUNK_T = EPT // ECHUNK

ROW_BLK = 256
N_BLKS = NPAD // ROW_BLK
COL_BLK = 1024
NT_BLKS = (NT + COL_BLK - 1) // COL_BLK

_mesh = plsc.VectorSubcoreMesh(core_axis_name="c", subcore_axis_name="s")
_sc_params = pltpu.CompilerParams(use_tc_tiling_on_sc=False)


def _xy_body(x_hbm, y_hbm, emb_hbm, xy_hbm, idxv, yv, gbuf, rowsa, rowsb):
    cid = lax.axis_index("c")
    sid = lax.axis_index("s")
    wid = sid * 2 + cid
    pltpu.sync_copy(x_hbm.at[pl.ds(wid * BPW, BPW)], idxv)
    pltpu.sync_copy(y_hbm.at[pl.ds(wid * BPW, BPW)], yv)
    for j in range(BPW // 80):
        pltpu.sync_copy(emb_hbm.at[idxv.at[pl.ds(j * 80, 80)]],
                        gbuf.at[pl.ds(j * 80, 80)])
    lane = lax.iota(jnp.int32, 16)
    zero16 = jnp.zeros((16,), jnp.float32)

    def body(g, _):
        y16 = yv[pl.ds(g * 16, 16)]
        for l in range(16):
            r = g * 16 + l
            for c in range(NCHH):
                rowsa[r, pl.ds(c * 16, 16)] = gbuf[r, pl.ds(c * 16, 16)]
            for c in range(3):
                rowsb[r, pl.ds(c * 16, 16)] = gbuf[r, pl.ds(80 + c * 16, 16)]
            rowsb[r, pl.ds(48, 16)] = jnp.where(
                lane == 0, jnp.full((16,), y16[l]), 0.0)
            rowsb[r, pl.ds(64, 16)] = zero16
        return _

    lax.fori_loop(0, BPW // 16, body, None)
    pltpu.sync_copy(rowsa, xy_hbm.at[pl.ds(wid * BPW, BPW)])
    pltpu.sync_copy(rowsb, xy_hbm.at[pl.ds(NPAD + wid * BPW, BPW)])


_xy_kernel = functools.partial(
    pl.kernel,
    out_type=jax.ShapeDtypeStruct((2 * NPAD, HALF), jnp.float32),
    mesh=_mesh,
    scratch_types=[
        pltpu.VMEM((BPW,), jnp.int32),
        pltpu.VMEM((BPW,), jnp.float32),
        pltpu.VMEM((BPW, EMB), jnp.float32),
        pltpu.VMEM((BPW, HALF), jnp.float32),
        pltpu.VMEM((BPW, HALF), jnp.float32),
    ],
    compiler_params=_sc_params,
)(_xy_body)


def _edge_body(xy_hbm, src_hbm, dst_hbm, ea_hbm, w_hbm, b_hbm, agg_hbm,
               srcv, dstv, eav, rows, wv, bv, aggs,
               isem0, isem1, gsem0, gsem1, ssem0, ssem1):
    cid = lax.axis_index("c")
    sid = lax.axis_index("s")
    pltpu.sync_copy(w_hbm.at[cid], wv)
    pltpu.sync_copy(b_hbm.at[cid], bv)
    isem = (isem0, isem1)
    gsem = (gsem0, gsem1)
    ssem = (ssem0, ssem1)

    def zbody(r, _):
        for c in range(NCHH):
            rows[0, r, pl.ds(c * 16, 16)] = jnp.zeros((16,), jnp.float32)
        return _

    lax.fori_loop(0, ECHUNK, zbody, None)
    trows = NPAD // 16
    for p in range(trows // ECHUNK):
        pltpu.sync_copy(rows.at[0],
                        aggs.at[pl.ds(sid * trows + p * ECHUNK, ECHUNK)])
    plsc.subcore_barrier()

    wvec = [wv[pl.ds(c * 16, 16)] for c in range(NCHH)]
    bvec = [bv[pl.ds(c * 16, 16)] for c in range(NCHH)]

    def idx_copies(i, b):
        rbase = sid * (EPT // SUB) + i * NSUB
        base = sid * EPT + i * ECHUNK
        return (
            pltpu.make_async_copy(src_hbm.at[cid, pl.ds(rbase, NSUB)],
                                  srcv.at[b], isem[b]),
            pltpu.make_async_copy(dst_hbm.at[pl.ds(rbase, NSUB)],
                                  dstv.at[b], isem[b]),
            pltpu.make_async_copy(ea_hbm.at[pl.ds(base, ECHUNK)],
                                  eav.at[b], isem[b]),
        )

    def gather_start(b):
        for j in range(NSUB):
            pltpu.async_copy(xy_hbm.at[srcv.at[b, j]],
                             rows.at[b, pl.ds(j * SUB, SUB)], gsem[b])

    def gather_wait(b):
        for j in range(NSUB):
            pltpu.make_async_copy(xy_hbm.at[srcv.at[b, j]],
                                  rows.at[b, pl.ds(j * SUB, SUB)],
                                  gsem[b]).wait()

    def scatter_start(b):
        for j in range(NSUB):
            pltpu.async_copy(rows.at[b, pl.ds(j * SUB, SUB)],
                             aggs.at[dstv.at[b, j]], ssem[b], add=True)

    def scatter_wait(b):
        for j in range(NSUB):
            pltpu.make_async_copy(rows.at[b, pl.ds(j * SUB, SUB)],
                                  aggs.at[dstv.at[b, j]], ssem[b]).wait()

    def start(copies):
        for c in copies:
            c.start()

    def wait(copies):
        for c in copies:
            c.wait()

    def compute(b):
        def ebody(g, _):
            ea16 = eav[b, pl.ds(g * 16, 16)]
            for l in range(16):
                rk = g * 16 + l
                eab = jnp.full((16,), ea16[l])
                for c in range(NCHH):
                    r = rows[b, rk, pl.ds(c * 16, 16)]
                    rows[b, rk, pl.ds(c * 16, 16)] = jnp.maximum(
                        r + (eab * wvec[c] + bvec[c]), 0.0)
            return _

        lax.fori_loop(0, ECHUNK // 16, ebody, None)

    start(idx_copies(0, 0))
    start(idx_copies(1, 1))
    wait(idx_copies(0, 0))
    gather_start(0)

    def step(i, b, nb):
        @pl.when(i + 1 < NCHUNK_T)
        def _():
            wait(idx_copies(i + 1, nb))

        @pl.when(i >= 1)
        def _():
            scatter_wait(nb)

        @pl.when(i + 1 < NCHUNK_T)
        def _():
            gather_start(nb)

        gather_wait(b)
        compute(b)

        @pl.when(i + 2 < NCHUNK_T)
        def _():
            start(idx_copies(i + 2, b))

        scatter_start(b)

    def pair(it, _):
        step(2 * it, 0, 1)
        step(2 * it + 1, 1, 0)
        return _

    lax.fori_loop(0, NCHUNK_T // 2, pair, None)
    scatter_wait((NCHUNK_T - 1) % 2)
    plsc.subcore_barrier()
    trows2 = NPAD // 16
    pltpu.sync_copy(aggs.at[pl.ds(sid * trows2, trows2)],
                    agg_hbm.at[cid, pl.ds(sid * trows2, trows2)])


_edge_kernel = functools.partial(
    pl.kernel,
    out_type=jax.ShapeDtypeStruct((2, NPAD, HALF), jnp.float32),
    mesh=_mesh,
    scratch_types=[
        pltpu.VMEM((2, NSUB, SUB), jnp.int32),
        pltpu.VMEM((2, NSUB, SUB), jnp.int32),
        pltpu.VMEM((2, ECHUNK), jnp.float32),
        pltpu.VMEM((2, ECHUNK, HALF), jnp.float32),
        pltpu.VMEM((HALF,), jnp.float32),
        pltpu.VMEM((HALF,), jnp.float32),
        pltpu.VMEM_SHARED((NPAD, HALF), jnp.float32),
        pltpu.SemaphoreType.DMA,
        pltpu.SemaphoreType.DMA,
        pltpu.SemaphoreType.DMA,
        pltpu.SemaphoreType.DMA,
        pltpu.SemaphoreType.DMA,
        pltpu.SemaphoreType.DMA,
    ],
    compiler_params=_sc_params,
)(_edge_body)


def _mlp_pool_body(xya_ref, xyb_ref, aga_ref, agb_ref, w1a_ref, w1b_ref,
                   b1_ref, w2_ref, b2_ref, batch_ref, hg_ref):
    i = pl.program_id(0)
    oa = xya_ref[0] + aga_ref[0]
    ob = xyb_ref[0] + agb_ref[0]
    h = jnp.maximum(
        jnp.dot(oa, w1a_ref[...], preferred_element_type=jnp.float32)
        + jnp.dot(ob, w1b_ref[...], preferred_element_type=jnp.float32)
        + b1_ref[...], 0.0)
    h = jnp.maximum(jnp.dot(h, w2_ref[...], preferred_element_type=jnp.float32)
                    + b2_ref[...], 0.0)
    b = batch_ref[0, 0, :]
    onehot_t = (lax.broadcasted_iota(jnp.int32, (NG, ROW_BLK), 0)
                == b[None, :]).astype(jnp.float32)

    @pl.when(i == 0)
    def _():
        hg_ref[...] = jnp.zeros_like(hg_ref)

    hg_ref[...] += jnp.dot(onehot_t, h, preferred_element_type=jnp.float32)


def _heads_body(hg_ref, wf1_ref, bf1_ref, wb1_ref, bb1_ref,
                wf2_ref, bf2_ref, wb2_ref, bb2_ref,
                f_ref, bck_ref, hf_s, hb_s):
    i = pl.program_id(0)

    @pl.when(i == 0)
    def _():
        hg = hg_ref[...]
        hf_s[...] = jnp.maximum(
            jnp.dot(hg, wf1_ref[...], preferred_element_type=jnp.float32)
            + bf1_ref[...], 0.0)
        hb_s[...] = jnp.maximum(
            jnp.dot(hg, wb1_ref[...], preferred_element_type=jnp.float32)
            + bb1_ref[...], 0.0)

    f_ref[...] = jnp.dot(hf_s[...], wf2_ref[...],
                         preferred_element_type=jnp.float32) + bf2_ref[...]
    bck_ref[...] = jnp.dot(hb_s[...], wb2_ref[...],
                           preferred_element_type=jnp.float32) + bb2_ref[...]


def kernel(x, y, edge_index, edge_attr, batch, emb, We, be, W1, b1, W2, b2,
           Wf1, bf1, Wf2, bf2, Wb1, bb1, Wb2, bb2):
    x_pad = jnp.zeros((NPAD,), jnp.int32).at[:N].set(x[:, 0].astype(jnp.int32))
    y_pad = jnp.zeros((NPAD,), jnp.float32).at[:N].set(y)
    xycat = _xy_kernel(x_pad, y_pad, emb)

    src = edge_index[0].astype(jnp.int32)
    src2 = jnp.stack([src, src + NPAD]).reshape(2, E // SUB, SUB)
    dst = edge_index[1].astype(jnp.int32).reshape(E // SUB, SUB)
    ea = edge_attr[:, 0]
    wrow = jnp.zeros((DPAD,), jnp.float32).at[:DIN].set(We[0]).reshape(2, HALF)
    brow = jnp.zeros((DPAD,), jnp.float32).at[:DIN].set(be).reshape(2, HALF)
    agg2 = _edge_kernel(xycat, src2, dst, ea, wrow, brow)

    W1p = jnp.zeros((DPAD, DH), jnp.float32).at[:DIN].set(W1)
    xy3 = xycat.reshape(2, NPAD, HALF)
    batch_pad = jnp.full((NPAD,), NG, jnp.int32).at[:N].set(
        batch.astype(jnp.int32))
    batch3 = batch_pad.reshape(N_BLKS, 1, ROW_BLK)

    hg = pl.pallas_call(
        _mlp_pool_body,
        grid=(N_BLKS,),
        in_specs=[
            pl.BlockSpec((1, ROW_BLK, HALF), lambda i: (0, i, 0)),
            pl.BlockSpec((1, ROW_BLK, HALF), lambda i: (1, i, 0)),
            pl.BlockSpec((1, ROW_BLK, HALF), lambda i: (0, i, 0)),
            pl.BlockSpec((1, ROW_BLK, HALF), lambda i: (1, i, 0)),
            pl.BlockSpec((HALF, DH), lambda i: (0, 0)),
            pl.BlockSpec((HALF, DH), lambda i: (0, 0)),
            pl.BlockSpec((1, DH), lambda i: (0, 0)),
            pl.BlockSpec((DH, DH), lambda i: (0, 0)),
            pl.BlockSpec((1, DH), lambda i: (0, 0)),
            pl.BlockSpec((1, 1, ROW_BLK), lambda i: (i, 0, 0)),
        ],
        out_specs=pl.BlockSpec((NG, DH), lambda i: (0, 0)),
        out_shape=jax.ShapeDtypeStruct((NG, DH), jnp.float32),
    )(xy3, xy3, agg2, agg2, W1p[:HALF], W1p[HALF:],
      b1.reshape(1, DH), W2, b2.reshape(1, DH), batch3)

    f, bck = pl.pallas_call(
        _heads_body,
        grid=(NT_BLKS,),
        in_specs=[
            pl.BlockSpec((NG, DH), lambda i: (0, 0)),
            pl.BlockSpec((DH, DH), lambda i: (0, 0)),
            pl.BlockSpec((1, DH), lambda i: (0, 0)),
            pl.BlockSpec((DH, DH), lambda i: (0, 0)),
            pl.BlockSpec((1, DH), lambda i: (0, 0)),
            pl.BlockSpec((DH, COL_BLK), lambda i: (0, i)),
            pl.BlockSpec((1, COL_BLK), lambda i: (0, i)),
            pl.BlockSpec((DH, COL_BLK), lambda i: (0, i)),
            pl.BlockSpec((1, COL_BLK), lambda i: (0, i)),
        ],
        out_specs=[
            pl.BlockSpec((NG, COL_BLK), lambda i: (0, i)),
            pl.BlockSpec((NG, COL_BLK), lambda i: (0, i)),
        ],
        out_shape=[
            jax.ShapeDtypeStruct((NG, NT), jnp.float32),
            jax.ShapeDtypeStruct((NG, NT), jnp.float32),
        ],
        scratch_shapes=[
            pltpu.VMEM((NG, DH), jnp.float32),
            pltpu.VMEM((NG, DH), jnp.float32),
        ],
    )(hg, Wf1, bf1.reshape(1, DH), Wb1, bb1.reshape(1, DH),
      Wf2, bf2.reshape(1, NT), Wb2, bb2.reshape(1, NT))

    return (f, bck)
